# baseline (speedup 1.0000x reference)
import functools

import jax
import jax.numpy as jnp
from jax import lax
from jax.experimental import pallas as pl
from jax.experimental.pallas import tpu as pltpu
from jax.experimental.pallas import tpu_sc as plsc

N = 10000
D = 128
NC = 2
NS = 16
NW = NC * NS
CK = 64
NCHUNK = 160
NBUF = 2
EPT = NCHUNK * CK
EP = NW * EPT
ROWS_PAD = 10112
RPT = ROWS_PAD // NS
TRASH = N
DEG_PAD = 10240

_mesh = plsc.VectorSubcoreMesh(core_axis_name="c", subcore_axis_name="s")


@functools.partial(
    pl.kernel,
    out_type=jax.ShapeDtypeStruct((NW, DEG_PAD), jnp.float32),
    mesh=_mesh,
    scratch_types=[
        pltpu.VMEM((EPT,), jnp.int32),
        pltpu.VMEM((DEG_PAD,), jnp.float32),
    ],
    compiler_params=pltpu.CompilerParams(needs_layout_passes=False),
)
def _deg_kernel(dst_hbm, out_hbm, idx_v, deg_v):
    c = lax.axis_index("c")
    s = lax.axis_index("s")
    wid = c * NS + s
    pltpu.sync_copy(dst_hbm.at[wid], idx_v)
    zeros = jnp.zeros((16,), jnp.float32)

    def zbody(i, carry):
        deg_v[pl.ds(i * 16, 16)] = zeros
        return carry

    lax.fori_loop(0, DEG_PAD // 16, zbody, 0)
    ones = jnp.ones((16,), jnp.float32)

    def body(i, carry):
        idx = idx_v[pl.ds(i * 16, 16)]
        plsc.addupdate_scatter(deg_v, [idx], ones)
        return carry

    lax.fori_loop(0, EPT // 16, body, 0)
    pltpu.sync_copy(deg_v, out_hbm.at[wid])


@functools.partial(
    pl.kernel,
    out_type=jax.ShapeDtypeStruct((NC, N, D), jnp.float32),
    mesh=_mesh,
    scratch_types=[
        pltpu.VMEM((8, CK), jnp.int32),
        pltpu.VMEM((8, CK), jnp.int32),
        pltpu.VMEM((CK, D), jnp.float32),
        pltpu.VMEM((CK, D), jnp.float32),
        pltpu.VMEM_SHARED((ROWS_PAD, D), jnp.float32),
        pltpu.SemaphoreType.DMA,
        pltpu.SemaphoreType.DMA,
        pltpu.SemaphoreType.DMA,
        pltpu.SemaphoreType.DMA,
    ],
    compiler_params=pltpu.CompilerParams(needs_layout_passes=False),
)
def _agg_kernel(h_hbm, src_hbm, dst_hbm, out_hbm,
                src_v, dst_v, st0, st1, acc_sh, ga0, ga1, sa0, sa1):
    stages = [st0, st1]
    gsems = [ga0, ga1]
    ssems = [sa0, sa1]
    c = lax.axis_index("c")
    s = lax.axis_index("s")
    wid = c * NS + s
    base = s * RPT
    zeros = jnp.zeros((16,), jnp.float32)

    def zbody(i, carry):
        st0[i, pl.ds(0, 16)] = zeros
        st0[i, pl.ds(16, 16)] = zeros
        st0[i, pl.ds(32, 16)] = zeros
        st0[i, pl.ds(48, 16)] = zeros
        st0[i, pl.ds(64, 16)] = zeros
        st0[i, pl.ds(80, 16)] = zeros
        st0[i, pl.ds(96, 16)] = zeros
        st0[i, pl.ds(112, 16)] = zeros
        return carry

    lax.fori_loop(0, CK, zbody, 0)
    for k in range(RPT // CK):
        pltpu.sync_copy(st0, acc_sh.at[pl.ds(base + k * CK, CK)])
    rem = RPT - (RPT // CK) * CK
    if rem:
        pltpu.sync_copy(st0.at[pl.ds(0, rem)],
                        acc_sh.at[pl.ds(base + (RPT // CK) * CK, rem)])
    plsc.subcore_barrier()

    def body(gi, carry):
        g = gi * NBUF
        for b in range(NBUF):
            pltpu.make_async_copy(
                h_hbm.at[src_v.at[g + b]], stages[b], gsems[b]).wait()
            pltpu.async_copy(
                stages[b], acc_sh.at[dst_v.at[g + b]], ssems[b], add=True)
        for b in range(NBUF):
            pltpu.make_async_copy(
                stages[b], acc_sh.at[dst_v.at[g + b]], ssems[b]).wait()
            nxt = lax.rem(g + b + NBUF, 8)
            pltpu.async_copy(h_hbm.at[src_v.at[nxt]], stages[b], gsems[b])
        return carry

    for sect in range(NCHUNK // 8):
        pltpu.sync_copy(src_hbm.at[wid, pl.ds(sect * 8, 8)], src_v)
        pltpu.sync_copy(dst_hbm.at[wid, pl.ds(sect * 8, 8)], dst_v)
        for b in range(NBUF):
            pltpu.async_copy(h_hbm.at[src_v.at[b]], stages[b], gsems[b])
        lax.fori_loop(0, 8 // NBUF, body, 0)
        for b in range(NBUF):
            pltpu.make_async_copy(
                h_hbm.at[src_v.at[b]], stages[b], gsems[b]).wait()
    plsc.subcore_barrier()
    last = N - (NS - 1) * RPT

    @pl.when(s < NS - 1)
    def _copy_full():
        pltpu.sync_copy(acc_sh.at[pl.ds(base, RPT)],
                        out_hbm.at[c, pl.ds(base, RPT)])

    @pl.when(s == NS - 1)
    def _copy_last():
        pltpu.sync_copy(acc_sh.at[pl.ds(base, last)],
                        out_hbm.at[c, pl.ds(base, last)])


BLK = 400
GRID = 25


def _tdis_body(degp_ref, dis_ref):
    deg = jnp.sum(degp_ref[...], axis=0) + 1.0
    dis_ref[...] = lax.rsqrt(deg)[:, None]


_tdis = pl.pallas_call(
    _tdis_body,
    in_specs=[pl.BlockSpec((NW, DEG_PAD), lambda: (0, 0))],
    out_specs=pl.BlockSpec((DEG_PAD, 1), lambda: (0, 0)),
    out_shape=jax.ShapeDtypeStruct((DEG_PAD, 1), jnp.float32),
)


def _t1_body(x_ref, w_ref, dis_ref, h_ref):
    h = jnp.dot(x_ref[...], w_ref[...],
                preferred_element_type=jnp.float32,
                precision=lax.Precision.HIGHEST)
    h_ref[...] = h * dis_ref[...]


_t1 = pl.pallas_call(
    _t1_body,
    grid=(GRID,),
    in_specs=[
        pl.BlockSpec((BLK, D), lambda i: (i, 0)),
        pl.BlockSpec((D, D), lambda i: (0, 0)),
        pl.BlockSpec((BLK, 1), lambda i: (i, 0)),
    ],
    out_specs=pl.BlockSpec((BLK, D), lambda i: (i, 0)),
    out_shape=jax.ShapeDtypeStruct((N, D), jnp.float32),
)


def _tmid_body(p_ref, hp_ref, dis_ref, b_ref, w_ref, out_ref):
    dis = dis_ref[...]
    h = dis * (p_ref[0] + p_ref[1] + hp_ref[...]) + b_ref[...]
    h = jnp.where(h >= 0, h, 0.01 * h)
    out_ref[...] = jnp.dot(h, w_ref[...],
                           preferred_element_type=jnp.float32,
                           precision=lax.Precision.HIGHEST) * dis


_tmid = pl.pallas_call(
    _tmid_body,
    grid=(GRID,),
    in_specs=[
        pl.BlockSpec((NC, BLK, D), lambda i: (0, i, 0)),
        pl.BlockSpec((BLK, D), lambda i: (i, 0)),
        pl.BlockSpec((BLK, 1), lambda i: (i, 0)),
        pl.BlockSpec((1, D), lambda i: (0, 0)),
        pl.BlockSpec((D, D), lambda i: (0, 0)),
    ],
    out_specs=pl.BlockSpec((BLK, D), lambda i: (i, 0)),
    out_shape=jax.ShapeDtypeStruct((N, D), jnp.float32),
)


def _t4_body(p_ref, zp_ref, dis_ref, b_ref, out_ref):
    out_ref[...] = (dis_ref[...] * (p_ref[0] + p_ref[1] + zp_ref[...])
                    + b_ref[...])


_t4 = pl.pallas_call(
    _t4_body,
    grid=(GRID,),
    in_specs=[
        pl.BlockSpec((NC, BLK, D), lambda i: (0, i, 0)),
        pl.BlockSpec((BLK, D), lambda i: (i, 0)),
        pl.BlockSpec((BLK, 1), lambda i: (i, 0)),
        pl.BlockSpec((1, D), lambda i: (0, 0)),
    ],
    out_specs=pl.BlockSpec((BLK, D), lambda i: (i, 0)),
    out_shape=jax.ShapeDtypeStruct((N, D), jnp.float32),
)


def kernel(x, W1, b1, W2, b2, Wmu, bmu, Wls, bls, edge_index):
    src = edge_index[0].astype(jnp.int32)
    dst = edge_index[1].astype(jnp.int32)
    e = src.shape[0]
    pad = EP - e
    src_p = jnp.concatenate([src, jnp.zeros((pad,), jnp.int32)])
    dst_p = jnp.concatenate([dst, jnp.full((pad,), TRASH, jnp.int32)])
    src3 = src_p.reshape(NW, NCHUNK, CK)
    dst3 = dst_p.reshape(NW, NCHUNK, CK)
    dst2 = dst_p.reshape(NW, EPT)
    degp = _deg_kernel(dst2)
    dis = _tdis(degp)[:N]
    h1p = _t1(x, W1, dis)
    p1 = _agg_kernel(h1p, src3, dst3)
    h2p = _tmid(p1, h1p, dis, b1.reshape(1, D), W2)
    p2 = _agg_kernel(h2p, src3, dst3)
    wcat = jnp.concatenate([Wmu, Wls], axis=1)
    bcat = jnp.concatenate([bmu, bls]).reshape(1, D)
    zp = _tmid(p2, h2p, dis, b2.reshape(1, D), wcat)
    p3 = _agg_kernel(zp, src3, dst3)
    out = _t4(p3, zp, dis, bcat)
    return (out[:, :64], out[:, 64:])


# speedup vs baseline: 1.2745x; 1.2745x over previous
import functools

import jax
import jax.numpy as jnp
from jax import lax
from jax.experimental import pallas as pl
from jax.experimental.pallas import tpu as pltpu
from jax.experimental.pallas import tpu_sc as plsc

N = 10000
D = 128
NC = 2
NS = 16
NW = NC * NS
CK = 64
NCHUNK = 160
NBUF = 2
EPT = NCHUNK * CK
EP = NW * EPT
ROWS_PAD = 10112
RPT = ROWS_PAD // NS
TRASH = N
DEG_PAD = 10240

_mesh = plsc.VectorSubcoreMesh(core_axis_name="c", subcore_axis_name="s")


@functools.partial(
    pl.kernel,
    out_type=jax.ShapeDtypeStruct((NW, DEG_PAD), jnp.float32),
    mesh=_mesh,
    scratch_types=[
        pltpu.VMEM((EPT,), jnp.int32),
        pltpu.VMEM((DEG_PAD,), jnp.float32),
    ],
    compiler_params=pltpu.CompilerParams(needs_layout_passes=False),
)
def _deg_kernel(dst_hbm, out_hbm, idx_v, deg_v):
    c = lax.axis_index("c")
    s = lax.axis_index("s")
    wid = c * NS + s
    pltpu.sync_copy(dst_hbm.at[wid], idx_v)
    zeros = jnp.zeros((16,), jnp.float32)

    def zbody(i, carry):
        deg_v[pl.ds(i * 16, 16)] = zeros
        return carry

    lax.fori_loop(0, DEG_PAD // 16, zbody, 0)
    ones = jnp.ones((16,), jnp.float32)

    def body(i, carry):
        idx = idx_v[pl.ds(i * 16, 16)]
        plsc.addupdate_scatter(deg_v, [idx], ones)
        return carry

    lax.fori_loop(0, EPT // 16, body, 0)
    pltpu.sync_copy(deg_v, out_hbm.at[wid])


@functools.partial(
    pl.kernel,
    out_type=jax.ShapeDtypeStruct((NC, N, D), jnp.float32),
    mesh=_mesh,
    scratch_types=[
        pltpu.VMEM((32, CK), jnp.int32),
        pltpu.VMEM((32, CK), jnp.int32),
        pltpu.VMEM((CK, D), jnp.float32),
        pltpu.VMEM((CK, D), jnp.float32),
        pltpu.VMEM((CK, D), jnp.float32),
        pltpu.VMEM((CK, D), jnp.float32),
        pltpu.VMEM_SHARED((ROWS_PAD, D), jnp.float32),
        pltpu.SemaphoreType.DMA,
        pltpu.SemaphoreType.DMA,
        pltpu.SemaphoreType.DMA,
        pltpu.SemaphoreType.DMA,
        pltpu.SemaphoreType.DMA,
        pltpu.SemaphoreType.DMA,
        pltpu.SemaphoreType.DMA,
        pltpu.SemaphoreType.DMA,
    ],
    compiler_params=pltpu.CompilerParams(needs_layout_passes=False),
)
def _agg_kernel(h_hbm, src_hbm, dst_hbm, out_hbm,
                src_v, dst_v, st0, st1, st2, st3, acc_sh,
                ga0, ga1, ga2, ga3, sa0, sa1, sa2, sa3):
    stages = [st0, st1, st2, st3]
    gsems = [ga0, ga1, ga2, ga3]
    ssems = [sa0, sa1, sa2, sa3]
    c = lax.axis_index("c")
    s = lax.axis_index("s")
    wid = c * NS + s
    base = s * RPT
    zeros = jnp.zeros((16,), jnp.float32)

    def zbody(i, carry):
        st0[i, pl.ds(0, 16)] = zeros
        st0[i, pl.ds(16, 16)] = zeros
        st0[i, pl.ds(32, 16)] = zeros
        st0[i, pl.ds(48, 16)] = zeros
        st0[i, pl.ds(64, 16)] = zeros
        st0[i, pl.ds(80, 16)] = zeros
        st0[i, pl.ds(96, 16)] = zeros
        st0[i, pl.ds(112, 16)] = zeros
        return carry

    lax.fori_loop(0, CK, zbody, 0)
    for k in range(RPT // CK):
        pltpu.sync_copy(st0, acc_sh.at[pl.ds(base + k * CK, CK)])
    rem = RPT - (RPT // CK) * CK
    if rem:
        pltpu.sync_copy(st0.at[pl.ds(0, rem)],
                        acc_sh.at[pl.ds(base + (RPT // CK) * CK, rem)])
    plsc.subcore_barrier()

    def body(bi, carry):
        cb = bi * 16
        gd = [pltpu.async_copy(h_hbm.at[src_v.at[cb + b]],
                               stages[b], gsems[b]) for b in range(4)]
        sd = [None] * 4
        for r in range(4):
            for b in range(4):
                gd[b].wait()
                sd[b] = pltpu.async_copy(
                    stages[b], acc_sh.at[dst_v.at[cb + r * 4 + b]],
                    ssems[b], add=True)
            if r < 3:
                for b in range(4):
                    sd[b].wait()
                    gd[b] = pltpu.async_copy(
                        h_hbm.at[src_v.at[cb + (r + 1) * 4 + b]],
                        stages[b], gsems[b])
        for b in range(4):
            sd[b].wait()
        return carry

    for sect in range(NCHUNK // 32):
        pltpu.sync_copy(src_hbm.at[wid, pl.ds(sect * 32, 32)], src_v)
        pltpu.sync_copy(dst_hbm.at[wid, pl.ds(sect * 32, 32)], dst_v)
        lax.fori_loop(0, 2, body, 0)
    plsc.subcore_barrier()
    last = N - (NS - 1) * RPT

    @pl.when(s < NS - 1)
    def _copy_full():
        pltpu.sync_copy(acc_sh.at[pl.ds(base, RPT)],
                        out_hbm.at[c, pl.ds(base, RPT)])

    @pl.when(s == NS - 1)
    def _copy_last():
        pltpu.sync_copy(acc_sh.at[pl.ds(base, last)],
                        out_hbm.at[c, pl.ds(base, last)])


BLK = 400
GRID = 25


def _tdis_body(degp_ref, dis_ref):
    deg = jnp.sum(degp_ref[...], axis=0) + 1.0
    dis_ref[...] = lax.rsqrt(deg)[:, None]


_tdis = pl.pallas_call(
    _tdis_body,
    in_specs=[pl.BlockSpec((NW, DEG_PAD), lambda: (0, 0))],
    out_specs=pl.BlockSpec((DEG_PAD, 1), lambda: (0, 0)),
    out_shape=jax.ShapeDtypeStruct((DEG_PAD, 1), jnp.float32),
)


def _t1_body(x_ref, w_ref, dis_ref, h_ref):
    h = jnp.dot(x_ref[...], w_ref[...],
                preferred_element_type=jnp.float32,
                precision=lax.Precision.HIGHEST)
    h_ref[...] = h * dis_ref[...]


_t1 = pl.pallas_call(
    _t1_body,
    grid=(GRID,),
    in_specs=[
        pl.BlockSpec((BLK, D), lambda i: (i, 0)),
        pl.BlockSpec((D, D), lambda i: (0, 0)),
        pl.BlockSpec((BLK, 1), lambda i: (i, 0)),
    ],
    out_specs=pl.BlockSpec((BLK, D), lambda i: (i, 0)),
    out_shape=jax.ShapeDtypeStruct((N, D), jnp.float32),
)


def _tmid_body(p_ref, hp_ref, dis_ref, b_ref, w_ref, out_ref):
    dis = dis_ref[...]
    h = dis * (p_ref[0] + p_ref[1] + hp_ref[...]) + b_ref[...]
    h = jnp.where(h >= 0, h, 0.01 * h)
    out_ref[...] = jnp.dot(h, w_ref[...],
                           preferred_element_type=jnp.float32,
                           precision=lax.Precision.HIGHEST) * dis


_tmid = pl.pallas_call(
    _tmid_body,
    grid=(GRID,),
    in_specs=[
        pl.BlockSpec((NC, BLK, D), lambda i: (0, i, 0)),
        pl.BlockSpec((BLK, D), lambda i: (i, 0)),
        pl.BlockSpec((BLK, 1), lambda i: (i, 0)),
        pl.BlockSpec((1, D), lambda i: (0, 0)),
        pl.BlockSpec((D, D), lambda i: (0, 0)),
    ],
    out_specs=pl.BlockSpec((BLK, D), lambda i: (i, 0)),
    out_shape=jax.ShapeDtypeStruct((N, D), jnp.float32),
)


def _t4_body(p_ref, zp_ref, dis_ref, b_ref, out_ref):
    out_ref[...] = (dis_ref[...] * (p_ref[0] + p_ref[1] + zp_ref[...])
                    + b_ref[...])


_t4 = pl.pallas_call(
    _t4_body,
    grid=(GRID,),
    in_specs=[
        pl.BlockSpec((NC, BLK, D), lambda i: (0, i, 0)),
        pl.BlockSpec((BLK, D), lambda i: (i, 0)),
        pl.BlockSpec((BLK, 1), lambda i: (i, 0)),
        pl.BlockSpec((1, D), lambda i: (0, 0)),
    ],
    out_specs=pl.BlockSpec((BLK, D), lambda i: (i, 0)),
    out_shape=jax.ShapeDtypeStruct((N, D), jnp.float32),
)


def kernel(x, W1, b1, W2, b2, Wmu, bmu, Wls, bls, edge_index):
    src = edge_index[0].astype(jnp.int32)
    dst = edge_index[1].astype(jnp.int32)
    e = src.shape[0]
    pad = EP - e
    src_p = jnp.concatenate([src, jnp.zeros((pad,), jnp.int32)])
    dst_p = jnp.concatenate([dst, jnp.full((pad,), TRASH, jnp.int32)])
    src3 = src_p.reshape(NW, NCHUNK, CK)
    dst3 = dst_p.reshape(NW, NCHUNK, CK)
    dst2 = dst_p.reshape(NW, EPT)
    degp = _deg_kernel(dst2)
    dis = _tdis(degp)[:N]
    h1p = _t1(x, W1, dis)
    p1 = _agg_kernel(h1p, src3, dst3)
    h2p = _tmid(p1, h1p, dis, b1.reshape(1, D), W2)
    p2 = _agg_kernel(h2p, src3, dst3)
    wcat = jnp.concatenate([Wmu, Wls], axis=1)
    bcat = jnp.concatenate([bmu, bls]).reshape(1, D)
    zp = _tmid(p2, h2p, dis, b2.reshape(1, D), wcat)
    p3 = _agg_kernel(zp, src3, dst3)
    out = _t4(p3, zp, dis, bcat)
    return (out[:, :64], out[:, 64:])


# speedup vs baseline: 1.3820x; 1.0844x over previous
import functools

import jax
import jax.numpy as jnp
from jax import lax
from jax.experimental import pallas as pl
from jax.experimental.pallas import tpu as pltpu
from jax.experimental.pallas import tpu_sc as plsc

N = 10000
D = 128
NC = 2
NS = 16
NW = NC * NS
CK = 128
NCHUNK = 80
NBUF = 2
EPT = NCHUNK * CK
EP = NW * EPT
ROWS_PAD = 10112
RPT = ROWS_PAD // NS
TRASH = N
DEG_PAD = 10240

_mesh = plsc.VectorSubcoreMesh(core_axis_name="c", subcore_axis_name="s")


@functools.partial(
    pl.kernel,
    out_type=jax.ShapeDtypeStruct((NW, DEG_PAD), jnp.float32),
    mesh=_mesh,
    scratch_types=[
        pltpu.VMEM((EPT,), jnp.int32),
        pltpu.VMEM((DEG_PAD,), jnp.float32),
    ],
    compiler_params=pltpu.CompilerParams(needs_layout_passes=False),
)
def _deg_kernel(dst_hbm, out_hbm, idx_v, deg_v):
    c = lax.axis_index("c")
    s = lax.axis_index("s")
    wid = c * NS + s
    pltpu.sync_copy(dst_hbm.at[wid], idx_v)
    zeros = jnp.zeros((16,), jnp.float32)

    def zbody(i, carry):
        deg_v[pl.ds(i * 16, 16)] = zeros
        return carry

    lax.fori_loop(0, DEG_PAD // 16, zbody, 0)
    ones = jnp.ones((16,), jnp.float32)

    def body(i, carry):
        idx = idx_v[pl.ds(i * 16, 16)]
        plsc.addupdate_scatter(deg_v, [idx], ones)
        return carry

    lax.fori_loop(0, EPT // 16, body, 0)
    pltpu.sync_copy(deg_v, out_hbm.at[wid])


@functools.partial(
    pl.kernel,
    out_type=jax.ShapeDtypeStruct((NC, N, D), jnp.float32),
    mesh=_mesh,
    scratch_types=[
        pltpu.VMEM((NCHUNK // 2, CK), jnp.int32),
        pltpu.VMEM((NCHUNK // 2, CK), jnp.int32),
        pltpu.VMEM((CK, D), jnp.float32),
        pltpu.VMEM((CK, D), jnp.float32),
        pltpu.VMEM_SHARED((ROWS_PAD, D), jnp.float32),
        pltpu.SemaphoreType.DMA,
    ],
    compiler_params=pltpu.CompilerParams(needs_layout_passes=False),
)
def _agg_kernel(h_hbm, src_hbm, dst_hbm, out_hbm,
                src_v, dst_v, st0, st1, acc_sh, gsem):
    c = lax.axis_index("c")
    s = lax.axis_index("s")
    wid = c * NS + s
    base = s * RPT
    zeros = jnp.zeros((16,), jnp.float32)

    def zbody(i, carry):
        for j in range(D // 16):
            st0[i, pl.ds(j * 16, 16)] = zeros
        return carry

    lax.fori_loop(0, CK, zbody, 0)
    for k in range(RPT // CK):
        pltpu.sync_copy(st0, acc_sh.at[pl.ds(base + k * CK, CK)])
    rem = RPT - (RPT // CK) * CK
    if rem:
        pltpu.sync_copy(st0.at[pl.ds(0, rem)],
                        acc_sh.at[pl.ds(base + (RPT // CK) * CK, rem)])
    plsc.subcore_barrier()

    def body(gi, carry):
        g = gi * 2
        d0 = pltpu.async_copy(h_hbm.at[src_v.at[g]], st0, gsem)
        d1 = pltpu.async_copy(h_hbm.at[src_v.at[g + 1]], st1, gsem)
        d0.wait()
        pltpu.sync_copy(st0, acc_sh.at[dst_v.at[g]], add=True)
        d1.wait()
        pltpu.sync_copy(st1, acc_sh.at[dst_v.at[g + 1]], add=True)
        return carry

    half = NCHUNK // 2
    for sect in range(2):
        pltpu.sync_copy(src_hbm.at[wid, pl.ds(sect * half, half)], src_v)
        pltpu.sync_copy(dst_hbm.at[wid, pl.ds(sect * half, half)], dst_v)
        lax.fori_loop(0, half // 2, body, 0)
    plsc.subcore_barrier()
    last = N - (NS - 1) * RPT

    @pl.when(s < NS - 1)
    def _copy_full():
        pltpu.sync_copy(acc_sh.at[pl.ds(base, RPT)],
                        out_hbm.at[c, pl.ds(base, RPT)])

    @pl.when(s == NS - 1)
    def _copy_last():
        pltpu.sync_copy(acc_sh.at[pl.ds(base, last)],
                        out_hbm.at[c, pl.ds(base, last)])


BLK = 400
GRID = 25


def _tdis_body(degp_ref, dis_ref):
    deg = jnp.sum(degp_ref[...], axis=0) + 1.0
    dis_ref[...] = lax.rsqrt(deg)[:, None]


_tdis = pl.pallas_call(
    _tdis_body,
    in_specs=[pl.BlockSpec((NW, DEG_PAD), lambda: (0, 0))],
    out_specs=pl.BlockSpec((DEG_PAD, 1), lambda: (0, 0)),
    out_shape=jax.ShapeDtypeStruct((DEG_PAD, 1), jnp.float32),
)


def _t1_body(x_ref, w_ref, dis_ref, h_ref):
    h = jnp.dot(x_ref[...], w_ref[...],
                preferred_element_type=jnp.float32,
                precision=lax.Precision.HIGHEST)
    h_ref[...] = h * dis_ref[...]


_t1 = pl.pallas_call(
    _t1_body,
    grid=(GRID,),
    in_specs=[
        pl.BlockSpec((BLK, D), lambda i: (i, 0)),
        pl.BlockSpec((D, D), lambda i: (0, 0)),
        pl.BlockSpec((BLK, 1), lambda i: (i, 0)),
    ],
    out_specs=pl.BlockSpec((BLK, D), lambda i: (i, 0)),
    out_shape=jax.ShapeDtypeStruct((N, D), jnp.float32),
)


def _tmid_body(p_ref, hp_ref, dis_ref, b_ref, w_ref, out_ref):
    dis = dis_ref[...]
    h = dis * (p_ref[0] + p_ref[1] + hp_ref[...]) + b_ref[...]
    h = jnp.where(h >= 0, h, 0.01 * h)
    out_ref[...] = jnp.dot(h, w_ref[...],
                           preferred_element_type=jnp.float32,
                           precision=lax.Precision.HIGHEST) * dis


_tmid = pl.pallas_call(
    _tmid_body,
    grid=(GRID,),
    in_specs=[
        pl.BlockSpec((NC, BLK, D), lambda i: (0, i, 0)),
        pl.BlockSpec((BLK, D), lambda i: (i, 0)),
        pl.BlockSpec((BLK, 1), lambda i: (i, 0)),
        pl.BlockSpec((1, D), lambda i: (0, 0)),
        pl.BlockSpec((D, D), lambda i: (0, 0)),
    ],
    out_specs=pl.BlockSpec((BLK, D), lambda i: (i, 0)),
    out_shape=jax.ShapeDtypeStruct((N, D), jnp.float32),
)


def _t4_body(p_ref, zp_ref, dis_ref, b_ref, out_ref):
    out_ref[...] = (dis_ref[...] * (p_ref[0] + p_ref[1] + zp_ref[...])
                    + b_ref[...])


_t4 = pl.pallas_call(
    _t4_body,
    grid=(GRID,),
    in_specs=[
        pl.BlockSpec((NC, BLK, D), lambda i: (0, i, 0)),
        pl.BlockSpec((BLK, D), lambda i: (i, 0)),
        pl.BlockSpec((BLK, 1), lambda i: (i, 0)),
        pl.BlockSpec((1, D), lambda i: (0, 0)),
    ],
    out_specs=pl.BlockSpec((BLK, D), lambda i: (i, 0)),
    out_shape=jax.ShapeDtypeStruct((N, D), jnp.float32),
)


def kernel(x, W1, b1, W2, b2, Wmu, bmu, Wls, bls, edge_index):
    src = edge_index[0].astype(jnp.int32)
    dst = edge_index[1].astype(jnp.int32)
    e = src.shape[0]
    pad = EP - e
    src_p = jnp.concatenate([src, jnp.zeros((pad,), jnp.int32)])
    dst_p = jnp.concatenate([dst, jnp.full((pad,), TRASH, jnp.int32)])
    src3 = src_p.reshape(NW, NCHUNK, CK)
    dst3 = dst_p.reshape(NW, NCHUNK, CK)
    dst2 = dst_p.reshape(NW, EPT)
    degp = _deg_kernel(dst2)
    dis = _tdis(degp)[:N]
    h1p = _t1(x, W1, dis)
    p1 = _agg_kernel(h1p, src3, dst3)
    h2p = _tmid(p1, h1p, dis, b1.reshape(1, D), W2)
    p2 = _agg_kernel(h2p, src3, dst3)
    wcat = jnp.concatenate([Wmu, Wls], axis=1)
    bcat = jnp.concatenate([bmu, bls]).reshape(1, D)
    zp = _tmid(p2, h2p, dis, b2.reshape(1, D), wcat)
    p3 = _agg_kernel(zp, src3, dst3)
    out = _t4(p3, zp, dis, bcat)
    return (out[:, :64], out[:, 64:])


# speedup vs baseline: 1.8405x; 1.3317x over previous
import functools

import jax
import jax.numpy as jnp
from jax import lax
from jax.experimental import pallas as pl
from jax.experimental.pallas import tpu as pltpu
from jax.experimental.pallas import tpu_sc as plsc

N = 10000
D = 128
NC = 2
NS = 16
NW = NC * NS
CK = 128
NC0 = 53
NC1 = 105
NCHUNK = 105
NBUF = 2
EP = NS * CK * (NC0 + NC1)
DEG_E = 327680
ROWS_PAD = 10112
RPT = ROWS_PAD // NS
TRASH = N
DEG_PAD = 10240

_mesh = plsc.VectorSubcoreMesh(core_axis_name="c", subcore_axis_name="s")


@functools.partial(
    pl.kernel,
    out_type=jax.ShapeDtypeStruct((NW, DEG_PAD), jnp.float32),
    mesh=_mesh,
    scratch_types=[
        pltpu.VMEM((DEG_E // NW,), jnp.int32),
        pltpu.VMEM((DEG_PAD,), jnp.float32),
    ],
    compiler_params=pltpu.CompilerParams(needs_layout_passes=False),
)
def _deg_kernel(dst_hbm, out_hbm, idx_v, deg_v):
    c = lax.axis_index("c")
    s = lax.axis_index("s")
    wid = c * NS + s
    pltpu.sync_copy(dst_hbm.at[wid], idx_v)
    zeros = jnp.zeros((16,), jnp.float32)

    def zbody(i, carry):
        deg_v[pl.ds(i * 16, 16)] = zeros
        return carry

    lax.fori_loop(0, DEG_PAD // 16, zbody, 0)
    ones = jnp.ones((16,), jnp.float32)

    def body(i, carry):
        idx = idx_v[pl.ds(i * 16, 16)]
        plsc.addupdate_scatter(deg_v, [idx], ones)
        return carry

    lax.fori_loop(0, DEG_E // NW // 16, body, 0)
    pltpu.sync_copy(deg_v, out_hbm.at[wid])


@functools.partial(
    pl.kernel,
    out_type=jax.ShapeDtypeStruct((NC, N, D), jnp.float32),
    mesh=_mesh,
    scratch_types=[
        pltpu.VMEM((NCHUNK, CK), jnp.int32),
        pltpu.VMEM((NCHUNK, CK), jnp.int32),
        pltpu.VMEM((CK, D), jnp.float32),
        pltpu.VMEM_SHARED((ROWS_PAD, D), jnp.float32),
        pltpu.SemaphoreType.DMA,
    ],
    compiler_params=pltpu.CompilerParams(needs_layout_passes=False),
)
def _agg_kernel(h_hbm, src_hbm, dst_hbm, out_hbm,
                src_v, dst_v, st0, acc_sh, gsem):
    c = lax.axis_index("c")
    s = lax.axis_index("s")
    wid = c * NS + s
    base = s * RPT
    pltpu.sync_copy(src_hbm.at[wid], src_v)
    pltpu.sync_copy(dst_hbm.at[wid], dst_v)
    zeros = jnp.zeros((16,), jnp.float32)

    def zbody(i, carry):
        for j in range(D // 16):
            st0[i, pl.ds(j * 16, 16)] = zeros
        return carry

    lax.fori_loop(0, CK, zbody, 0)
    for k in range(RPT // CK):
        pltpu.sync_copy(st0, acc_sh.at[pl.ds(base + k * CK, CK)])
    rem = RPT - (RPT // CK) * CK
    if rem:
        pltpu.sync_copy(st0.at[pl.ds(0, rem)],
                        acc_sh.at[pl.ds(base + (RPT // CK) * CK, rem)])
    plsc.subcore_barrier()

    def body(ci, carry):
        pltpu.async_copy(h_hbm.at[src_v.at[ci]], st0, gsem).wait()
        pltpu.sync_copy(st0, acc_sh.at[dst_v.at[ci]], add=True)
        return carry

    nchunk = jnp.where(c == 0, NC0, NC1)
    lax.fori_loop(0, nchunk, body, 0)
    plsc.subcore_barrier()
    last = N - (NS - 1) * RPT

    @pl.when(s < NS - 1)
    def _copy_full():
        pltpu.sync_copy(acc_sh.at[pl.ds(base, RPT)],
                        out_hbm.at[c, pl.ds(base, RPT)])

    @pl.when(s == NS - 1)
    def _copy_last():
        pltpu.sync_copy(acc_sh.at[pl.ds(base, last)],
                        out_hbm.at[c, pl.ds(base, last)])


BLK = 400
GRID = 25


def _tdis_body(degp_ref, dis_ref):
    deg = jnp.sum(degp_ref[...], axis=0) + 1.0
    dis_ref[...] = lax.rsqrt(deg)[:, None]


_tdis = pl.pallas_call(
    _tdis_body,
    in_specs=[pl.BlockSpec((NW, DEG_PAD), lambda: (0, 0))],
    out_specs=pl.BlockSpec((DEG_PAD, 1), lambda: (0, 0)),
    out_shape=jax.ShapeDtypeStruct((DEG_PAD, 1), jnp.float32),
)


def _t1_body(x_ref, w_ref, dis_ref, h_ref):
    h = jnp.dot(x_ref[...], w_ref[...],
                preferred_element_type=jnp.float32,
                precision=lax.Precision.HIGHEST)
    h_ref[...] = h * dis_ref[...]


_t1 = pl.pallas_call(
    _t1_body,
    grid=(GRID,),
    in_specs=[
        pl.BlockSpec((BLK, D), lambda i: (i, 0)),
        pl.BlockSpec((D, D), lambda i: (0, 0)),
        pl.BlockSpec((BLK, 1), lambda i: (i, 0)),
    ],
    out_specs=pl.BlockSpec((BLK, D), lambda i: (i, 0)),
    out_shape=jax.ShapeDtypeStruct((N, D), jnp.float32),
)


def _tmid_body(p_ref, hp_ref, dis_ref, b_ref, w_ref, out_ref):
    dis = dis_ref[...]
    h = dis * (p_ref[0] + p_ref[1] + hp_ref[...]) + b_ref[...]
    h = jnp.where(h >= 0, h, 0.01 * h)
    out_ref[...] = jnp.dot(h, w_ref[...],
                           preferred_element_type=jnp.float32,
                           precision=lax.Precision.HIGHEST) * dis


_tmid = pl.pallas_call(
    _tmid_body,
    grid=(GRID,),
    in_specs=[
        pl.BlockSpec((NC, BLK, D), lambda i: (0, i, 0)),
        pl.BlockSpec((BLK, D), lambda i: (i, 0)),
        pl.BlockSpec((BLK, 1), lambda i: (i, 0)),
        pl.BlockSpec((1, D), lambda i: (0, 0)),
        pl.BlockSpec((D, D), lambda i: (0, 0)),
    ],
    out_specs=pl.BlockSpec((BLK, D), lambda i: (i, 0)),
    out_shape=jax.ShapeDtypeStruct((N, D), jnp.float32),
)


def _t4_body(p_ref, zp_ref, dis_ref, b_ref, out_ref):
    out_ref[...] = (dis_ref[...] * (p_ref[0] + p_ref[1] + zp_ref[...])
                    + b_ref[...])


_t4 = pl.pallas_call(
    _t4_body,
    grid=(GRID,),
    in_specs=[
        pl.BlockSpec((NC, BLK, D), lambda i: (0, i, 0)),
        pl.BlockSpec((BLK, D), lambda i: (i, 0)),
        pl.BlockSpec((BLK, 1), lambda i: (i, 0)),
        pl.BlockSpec((1, D), lambda i: (0, 0)),
    ],
    out_specs=pl.BlockSpec((BLK, D), lambda i: (i, 0)),
    out_shape=jax.ShapeDtypeStruct((N, D), jnp.float32),
)


def kernel(x, W1, b1, W2, b2, Wmu, bmu, Wls, bls, edge_index):
    src = edge_index[0].astype(jnp.int32)
    dst = edge_index[1].astype(jnp.int32)
    e = src.shape[0]
    src_p = jnp.concatenate([src, jnp.zeros((EP - e,), jnp.int32)])
    dst_p = jnp.concatenate([dst, jnp.full((EP - e,), TRASH, jnp.int32)])
    n0 = NS * NC0 * CK
    s0 = src_p[:n0].reshape(NS, NC0, CK)
    s0 = jnp.pad(s0, ((0, 0), (0, NCHUNK - NC0), (0, 0)))
    d0 = dst_p[:n0].reshape(NS, NC0, CK)
    d0 = jnp.pad(d0, ((0, 0), (0, NCHUNK - NC0), (0, 0)),
                 constant_values=TRASH)
    s1 = src_p[n0:].reshape(NS, NC1, CK)
    d1 = dst_p[n0:].reshape(NS, NC1, CK)
    src3 = jnp.concatenate([s0, s1])
    dst3 = jnp.concatenate([d0, d1])
    dst2 = jnp.concatenate(
        [dst, jnp.full((DEG_E - e,), TRASH, jnp.int32)]).reshape(NW, -1)

    degp = _deg_kernel(dst2)
    dis = _tdis(degp)[:N]
    h1p = _t1(x, W1, dis)
    p1 = _agg_kernel(h1p, src3, dst3)
    h2p = _tmid(p1, h1p, dis, b1.reshape(1, D), W2)
    p2 = _agg_kernel(h2p, src3, dst3)
    wcat = jnp.concatenate([Wmu, Wls], axis=1)
    bcat = jnp.concatenate([bmu, bls]).reshape(1, D)
    zp = _tmid(p2, h2p, dis, b2.reshape(1, D), wcat)
    p3 = _agg_kernel(zp, src3, dst3)
    out = _t4(p3, zp, dis, bcat)
    return (out[:, :64], out[:, 64:])


# speedup vs baseline: 2.3308x; 1.2664x over previous
import functools

import jax
import jax.numpy as jnp
from jax import lax
from jax.experimental import pallas as pl
from jax.experimental.pallas import tpu as pltpu
from jax.experimental.pallas import tpu_sc as plsc

N = 10000
D = 128
NC = 2
NS = 16
NW = NC * NS
CK = 128
NC0 = 105
NC1 = 53
NCHUNK = 105
NBUF = 2
EP = NS * CK * (NC0 + NC1)
DEG_E = 327680
ROWS_PAD = 10112
RPT = ROWS_PAD // NS
TRASH = N
DEG_PAD = 10240

_mesh = plsc.VectorSubcoreMesh(core_axis_name="c", subcore_axis_name="s")


@functools.partial(
    pl.kernel,
    out_type=jax.ShapeDtypeStruct((NW, DEG_PAD), jnp.float32),
    mesh=_mesh,
    scratch_types=[
        pltpu.VMEM((DEG_E // NW,), jnp.int32),
        pltpu.VMEM((DEG_PAD,), jnp.float32),
    ],
    compiler_params=pltpu.CompilerParams(needs_layout_passes=False),
)
def _deg_kernel(dst_hbm, out_hbm, idx_v, deg_v):
    c = lax.axis_index("c")
    s = lax.axis_index("s")
    wid = c * NS + s
    pltpu.sync_copy(dst_hbm.at[wid], idx_v)
    zeros = jnp.zeros((16,), jnp.float32)

    def zbody(i, carry):
        deg_v[pl.ds(i * 16, 16)] = zeros
        return carry

    lax.fori_loop(0, DEG_PAD // 16, zbody, 0)
    ones = jnp.ones((16,), jnp.float32)

    def body(i, carry):
        idx = idx_v[pl.ds(i * 16, 16)]
        plsc.addupdate_scatter(deg_v, [idx], ones)
        return carry

    lax.fori_loop(0, DEG_E // NW // 16, body, 0)
    pltpu.sync_copy(deg_v, out_hbm.at[wid])


@functools.partial(
    pl.kernel,
    out_type=jax.ShapeDtypeStruct((NC, N, D), jnp.float32),
    mesh=_mesh,
    scratch_types=[
        pltpu.VMEM((NCHUNK, CK), jnp.int32),
        pltpu.VMEM((NCHUNK, CK), jnp.int32),
        pltpu.VMEM((CK, D), jnp.float32),
        pltpu.VMEM_SHARED((ROWS_PAD, D), jnp.float32),
        pltpu.SemaphoreType.DMA,
    ],
    compiler_params=pltpu.CompilerParams(needs_layout_passes=False),
)
def _agg_kernel(h_hbm, src_hbm, dst_hbm, out_hbm,
                src_v, dst_v, st0, acc_sh, gsem):
    c = lax.axis_index("c")
    s = lax.axis_index("s")
    wid = c * NS + s
    base = s * RPT
    pltpu.sync_copy(src_hbm.at[wid], src_v)
    pltpu.sync_copy(dst_hbm.at[wid], dst_v)
    zeros = jnp.zeros((16,), jnp.float32)

    def zbody(i, carry):
        for j in range(D // 16):
            st0[i, pl.ds(j * 16, 16)] = zeros
        return carry

    lax.fori_loop(0, CK, zbody, 0)
    for k in range(RPT // CK):
        pltpu.sync_copy(st0, acc_sh.at[pl.ds(base + k * CK, CK)])
    rem = RPT - (RPT // CK) * CK
    if rem:
        pltpu.sync_copy(st0.at[pl.ds(0, rem)],
                        acc_sh.at[pl.ds(base + (RPT // CK) * CK, rem)])
    plsc.subcore_barrier()

    def body(ci, carry):
        pltpu.async_copy(h_hbm.at[src_v.at[ci]], st0, gsem).wait()
        pltpu.sync_copy(st0, acc_sh.at[dst_v.at[ci]], add=True)
        return carry

    nchunk = jnp.where(c == 0, NC0, NC1)
    lax.fori_loop(0, nchunk, body, 0)
    plsc.subcore_barrier()
    last = N - (NS - 1) * RPT

    @pl.when(s < NS - 1)
    def _copy_full():
        pltpu.sync_copy(acc_sh.at[pl.ds(base, RPT)],
                        out_hbm.at[c, pl.ds(base, RPT)])

    @pl.when(s == NS - 1)
    def _copy_last():
        pltpu.sync_copy(acc_sh.at[pl.ds(base, last)],
                        out_hbm.at[c, pl.ds(base, last)])


BLK = 400
GRID = 25


def _tdis_body(degp_ref, dis_ref):
    deg = jnp.sum(degp_ref[...], axis=0) + 1.0
    dis_ref[...] = lax.rsqrt(deg)[:, None]


_tdis = pl.pallas_call(
    _tdis_body,
    in_specs=[pl.BlockSpec((NW, DEG_PAD), lambda: (0, 0))],
    out_specs=pl.BlockSpec((DEG_PAD, 1), lambda: (0, 0)),
    out_shape=jax.ShapeDtypeStruct((DEG_PAD, 1), jnp.float32),
)


def _t1_body(x_ref, w_ref, dis_ref, h_ref):
    h = jnp.dot(x_ref[...], w_ref[...],
                preferred_element_type=jnp.float32,
                precision=lax.Precision.HIGHEST)
    h_ref[...] = h * dis_ref[...]


_t1 = pl.pallas_call(
    _t1_body,
    grid=(GRID,),
    in_specs=[
        pl.BlockSpec((BLK, D), lambda i: (i, 0)),
        pl.BlockSpec((D, D), lambda i: (0, 0)),
        pl.BlockSpec((BLK, 1), lambda i: (i, 0)),
    ],
    out_specs=pl.BlockSpec((BLK, D), lambda i: (i, 0)),
    out_shape=jax.ShapeDtypeStruct((N, D), jnp.float32),
)


def _tmid_body(p_ref, hp_ref, dis_ref, b_ref, w_ref, out_ref):
    dis = dis_ref[...]
    h = dis * (p_ref[0] + p_ref[1] + hp_ref[...]) + b_ref[...]
    h = jnp.where(h >= 0, h, 0.01 * h)
    out_ref[...] = jnp.dot(h, w_ref[...],
                           preferred_element_type=jnp.float32,
                           precision=lax.Precision.HIGHEST) * dis


_tmid = pl.pallas_call(
    _tmid_body,
    grid=(GRID,),
    in_specs=[
        pl.BlockSpec((NC, BLK, D), lambda i: (0, i, 0)),
        pl.BlockSpec((BLK, D), lambda i: (i, 0)),
        pl.BlockSpec((BLK, 1), lambda i: (i, 0)),
        pl.BlockSpec((1, D), lambda i: (0, 0)),
        pl.BlockSpec((D, D), lambda i: (0, 0)),
    ],
    out_specs=pl.BlockSpec((BLK, D), lambda i: (i, 0)),
    out_shape=jax.ShapeDtypeStruct((N, D), jnp.float32),
)


def _t4_body(p_ref, zp_ref, dis_ref, b_ref, out_ref):
    out_ref[...] = (dis_ref[...] * (p_ref[0] + p_ref[1] + zp_ref[...])
                    + b_ref[...])


_t4 = pl.pallas_call(
    _t4_body,
    grid=(GRID,),
    in_specs=[
        pl.BlockSpec((NC, BLK, D), lambda i: (0, i, 0)),
        pl.BlockSpec((BLK, D), lambda i: (i, 0)),
        pl.BlockSpec((BLK, 1), lambda i: (i, 0)),
        pl.BlockSpec((1, D), lambda i: (0, 0)),
    ],
    out_specs=pl.BlockSpec((BLK, D), lambda i: (i, 0)),
    out_shape=jax.ShapeDtypeStruct((N, D), jnp.float32),
)


def kernel(x, W1, b1, W2, b2, Wmu, bmu, Wls, bls, edge_index):
    src = edge_index[0].astype(jnp.int32)
    dst = edge_index[1].astype(jnp.int32)
    e = src.shape[0]
    src_p = jnp.concatenate([src, jnp.zeros((EP - e,), jnp.int32)])
    dst_p = jnp.concatenate([dst, jnp.full((EP - e,), TRASH, jnp.int32)])
    n0 = NS * NC0 * CK
    s0 = jnp.pad(src_p[:n0].reshape(NS, NC0, CK),
                 ((0, 0), (0, NCHUNK - NC0), (0, 0)))
    d0 = jnp.pad(dst_p[:n0].reshape(NS, NC0, CK),
                 ((0, 0), (0, NCHUNK - NC0), (0, 0)), constant_values=TRASH)
    s1 = jnp.pad(src_p[n0:].reshape(NS, NC1, CK),
                 ((0, 0), (0, NCHUNK - NC1), (0, 0)))
    d1 = jnp.pad(dst_p[n0:].reshape(NS, NC1, CK),
                 ((0, 0), (0, NCHUNK - NC1), (0, 0)), constant_values=TRASH)
    src3 = jnp.concatenate([s0, s1])
    dst3 = jnp.concatenate([d0, d1])
    dst2 = jnp.concatenate(
        [dst, jnp.full((DEG_E - e,), TRASH, jnp.int32)]).reshape(NW, -1)

    degp = _deg_kernel(dst2)
    dis = _tdis(degp)[:N]
    h1p = _t1(x, W1, dis)
    p1 = _agg_kernel(h1p, src3, dst3)
    h2p = _tmid(p1, h1p, dis, b1.reshape(1, D), W2)
    p2 = _agg_kernel(h2p, src3, dst3)
    wcat = jnp.concatenate([Wmu, Wls], axis=1)
    bcat = jnp.concatenate([bmu, bls]).reshape(1, D)
    zp = _tmid(p2, h2p, dis, b2.reshape(1, D), wcat)
    p3 = _agg_kernel(zp, src3, dst3)
    out = _t4(p3, zp, dis, bcat)
    return (out[:, :64], out[:, 64:])


# speedup vs baseline: 2.3315x; 1.0003x over previous
import functools

import jax
import jax.numpy as jnp
from jax import lax
from jax.experimental import pallas as pl
from jax.experimental.pallas import tpu as pltpu
from jax.experimental.pallas import tpu_sc as plsc

N = 10000
D = 128
NC = 2
NS = 16
NW = NC * NS
CK = 128
NC0 = 105
NC1 = 53
NCHUNK = 105
EP = NS * CK * (NC0 + NC1)
DEG_E = 327680
ROWS_PAD = 10112
RPT = ROWS_PAD // NS
TRASH = N
DEG_PAD = 10240

_mesh = plsc.VectorSubcoreMesh(core_axis_name="c", subcore_axis_name="s")


@functools.partial(
    pl.kernel,
    out_type=jax.ShapeDtypeStruct((NW, DEG_PAD), jnp.float32),
    mesh=_mesh,
    scratch_types=[
        pltpu.VMEM((DEG_E // NW,), jnp.int32),
        pltpu.VMEM((DEG_PAD,), jnp.float32),
    ],
    compiler_params=pltpu.CompilerParams(needs_layout_passes=False),
)
def _deg_kernel(dst_hbm, out_hbm, idx_v, deg_v):
    c = lax.axis_index("c")
    s = lax.axis_index("s")
    wid = c * NS + s
    pltpu.sync_copy(dst_hbm.at[wid], idx_v)
    zeros = jnp.zeros((16,), jnp.float32)

    def zbody(i, carry):
        deg_v[pl.ds(i * 16, 16)] = zeros
        return carry

    lax.fori_loop(0, DEG_PAD // 16, zbody, 0)
    ones = jnp.ones((16,), jnp.float32)

    def body(i, carry):
        idx = idx_v[pl.ds(i * 16, 16)]
        plsc.addupdate_scatter(deg_v, [idx], ones)
        return carry

    lax.fori_loop(0, DEG_E // NW // 16, body, 0)
    pltpu.sync_copy(deg_v, out_hbm.at[wid])


@functools.partial(
    pl.kernel,
    out_type=jax.ShapeDtypeStruct((NC, N, D), jnp.float32),
    mesh=_mesh,
    scratch_types=[
        pltpu.VMEM((NCHUNK, CK), jnp.int32),
        pltpu.VMEM((NCHUNK, CK), jnp.int32),
        pltpu.VMEM((CK, D), jnp.float32),
        pltpu.VMEM_SHARED((ROWS_PAD, D), jnp.float32),
        pltpu.SemaphoreType.DMA,
    ],
    compiler_params=pltpu.CompilerParams(needs_layout_passes=False),
)
def _agg_kernel(h_hbm, src_hbm, dst_hbm, out_hbm,
                src_v, dst_v, st0, acc_sh, gsem):
    c = lax.axis_index("c")
    s = lax.axis_index("s")
    wid = c * NS + s
    base = s * RPT
    pltpu.sync_copy(src_hbm.at[wid], src_v)
    pltpu.sync_copy(dst_hbm.at[wid], dst_v)
    zeros = jnp.zeros((16,), jnp.float32)

    def zbody(i, carry):
        for j in range(D // 16):
            st0[i, pl.ds(j * 16, 16)] = zeros
        return carry

    lax.fori_loop(0, CK, zbody, 0)
    for k in range(RPT // CK):
        pltpu.sync_copy(st0, acc_sh.at[pl.ds(base + k * CK, CK)])
    rem = RPT - (RPT // CK) * CK
    if rem:
        pltpu.sync_copy(st0.at[pl.ds(0, rem)],
                        acc_sh.at[pl.ds(base + (RPT // CK) * CK, rem)])
    plsc.subcore_barrier()

    def body(ci, carry):
        pltpu.async_copy(h_hbm.at[src_v.at[ci]], st0, gsem).wait()
        pltpu.sync_copy(st0, acc_sh.at[dst_v.at[ci]], add=True)
        return carry

    nchunk = jnp.where(c == 0, NC0, NC1)
    lax.fori_loop(0, nchunk, body, 0)
    plsc.subcore_barrier()
    last = N - (NS - 1) * RPT

    @pl.when(s < NS - 1)
    def _copy_full():
        pltpu.sync_copy(acc_sh.at[pl.ds(base, RPT)],
                        out_hbm.at[c, pl.ds(base, RPT)])

    @pl.when(s == NS - 1)
    def _copy_last():
        pltpu.sync_copy(acc_sh.at[pl.ds(base, last)],
                        out_hbm.at[c, pl.ds(base, last)])


BLK = 400
GRID = 25


def _tdis_body(degp_ref, dis_ref):
    deg = jnp.sum(degp_ref[...], axis=0) + 1.0
    dis_ref[...] = lax.rsqrt(deg)[:, None]


_tdis = pl.pallas_call(
    _tdis_body,
    in_specs=[pl.BlockSpec((NW, DEG_PAD), lambda: (0, 0))],
    out_specs=pl.BlockSpec((DEG_PAD, 1), lambda: (0, 0)),
    out_shape=jax.ShapeDtypeStruct((DEG_PAD, 1), jnp.float32),
)


def _t1_body(x_ref, w_ref, dis_ref, h_ref):
    h = jnp.dot(x_ref[...], w_ref[...],
                preferred_element_type=jnp.float32,
                precision=lax.Precision.HIGHEST)
    h_ref[...] = h * dis_ref[...]


_t1 = pl.pallas_call(
    _t1_body,
    grid=(GRID,),
    in_specs=[
        pl.BlockSpec((BLK, D), lambda i: (i, 0)),
        pl.BlockSpec((D, D), lambda i: (0, 0)),
        pl.BlockSpec((BLK, 1), lambda i: (i, 0)),
    ],
    out_specs=pl.BlockSpec((BLK, D), lambda i: (i, 0)),
    out_shape=jax.ShapeDtypeStruct((N, D), jnp.float32),
)


def _tmid_body(p_ref, hp_ref, dis_ref, b_ref, w_ref, out_ref):
    dis = dis_ref[...]
    h = dis * (p_ref[0] + p_ref[1] + hp_ref[...]) + b_ref[...]
    h = jnp.where(h >= 0, h, 0.01 * h)
    out_ref[...] = jnp.dot(h, w_ref[...],
                           preferred_element_type=jnp.float32,
                           precision=lax.Precision.HIGHEST) * dis


_tmid = pl.pallas_call(
    _tmid_body,
    grid=(GRID,),
    in_specs=[
        pl.BlockSpec((NC, BLK, D), lambda i: (0, i, 0)),
        pl.BlockSpec((BLK, D), lambda i: (i, 0)),
        pl.BlockSpec((BLK, 1), lambda i: (i, 0)),
        pl.BlockSpec((1, D), lambda i: (0, 0)),
        pl.BlockSpec((D, D), lambda i: (0, 0)),
    ],
    out_specs=pl.BlockSpec((BLK, D), lambda i: (i, 0)),
    out_shape=jax.ShapeDtypeStruct((N, D), jnp.float32),
)


def _t4_body(p_ref, zp_ref, dis_ref, b_ref, out_ref):
    out_ref[...] = (dis_ref[...] * (p_ref[0] + p_ref[1] + zp_ref[...])
                    + b_ref[...])


_t4 = pl.pallas_call(
    _t4_body,
    grid=(GRID,),
    in_specs=[
        pl.BlockSpec((NC, BLK, D), lambda i: (0, i, 0)),
        pl.BlockSpec((BLK, D), lambda i: (i, 0)),
        pl.BlockSpec((BLK, 1), lambda i: (i, 0)),
        pl.BlockSpec((1, D), lambda i: (0, 0)),
    ],
    out_specs=pl.BlockSpec((BLK, D), lambda i: (i, 0)),
    out_shape=jax.ShapeDtypeStruct((N, D), jnp.float32),
)


def kernel(x, W1, b1, W2, b2, Wmu, bmu, Wls, bls, edge_index):
    src = edge_index[0].astype(jnp.int32)
    dst = edge_index[1].astype(jnp.int32)
    e = src.shape[0]
    src_p = jnp.concatenate([src, jnp.zeros((EP - e,), jnp.int32)])
    dst_p = jnp.concatenate([dst, jnp.full((EP - e,), TRASH, jnp.int32)])
    n0 = NS * NC0 * CK
    s0 = jnp.pad(src_p[:n0].reshape(NS, NC0, CK),
                 ((0, 0), (0, NCHUNK - NC0), (0, 0)))
    d0 = jnp.pad(dst_p[:n0].reshape(NS, NC0, CK),
                 ((0, 0), (0, NCHUNK - NC0), (0, 0)), constant_values=TRASH)
    s1 = jnp.pad(src_p[n0:].reshape(NS, NC1, CK),
                 ((0, 0), (0, NCHUNK - NC1), (0, 0)))
    d1 = jnp.pad(dst_p[n0:].reshape(NS, NC1, CK),
                 ((0, 0), (0, NCHUNK - NC1), (0, 0)), constant_values=TRASH)
    src3 = jnp.concatenate([s0, s1])
    dst3 = jnp.concatenate([d0, d1])
    dst2 = jnp.concatenate(
        [dst, jnp.full((DEG_E - e,), TRASH, jnp.int32)]).reshape(NW, -1)

    degp = _deg_kernel(dst2)
    dis = _tdis(degp)[:N]
    h1p = _t1(x, W1, dis)
    p1 = _agg_kernel(h1p, src3, dst3)
    h2p = _tmid(p1, h1p, dis, b1.reshape(1, D), W2)
    p2 = _agg_kernel(h2p, src3, dst3)
    wcat = jnp.concatenate([Wmu, Wls], axis=1)
    bcat = jnp.concatenate([bmu, bls]).reshape(1, D)
    zp = _tmid(p2, h2p, dis, b2.reshape(1, D), wcat)
    p3 = _agg_kernel(zp, src3, dst3)
    out = _t4(p3, zp, dis, bcat)
    return (out[:, :64], out[:, 64:])
